# baseline (speedup 1.0000x reference)
import functools

import jax
import jax.numpy as jnp
from jax import lax
from jax.experimental import pallas as pl
from jax.experimental.pallas import tpu as pltpu
from jax.experimental.pallas import tpu_sc as plsc

_B = 4
_S = 8192
_D = 128
_BT = _B * _S

_info = plsc.get_sparse_core_info()
_NC, _NS, _L = _info.num_cores, _info.num_subcores, _info.num_lanes
_NW = _NC * _NS
_BPW = _BT // _NW
_C = 128
_NCHUNK = _BPW // _C


@functools.partial(
    pl.kernel,
    mesh=plsc.VectorSubcoreMesh(core_axis_name="c", subcore_axis_name="s"),
    out_type=jax.ShapeDtypeStruct((_BT, _D), jnp.float32),
    scratch_types=[
        pltpu.VMEM((_C,), jnp.int32),
        pltpu.VMEM((_C, _D), jnp.float32),
        pltpu.VMEM((_C, _D), jnp.float32),
        pltpu.SemaphoreType.DMA,
    ],
)
def _emb_lookup(idx_hbm, table_hbm, pos_hbm, out_hbm, idx_v, tok_v, pos_v, gsem):
    wid = lax.axis_index("s") * _NC + lax.axis_index("c")
    base = wid * _BPW
    pos_base = lax.rem(base, _S)
    for j in range(_NCHUNK):
        row0 = base + j * _C
        prow0 = pos_base + j * _C
        pltpu.sync_copy(idx_hbm.at[pl.ds(row0, _C)], idx_v)
        gather = pltpu.async_copy(table_hbm.at[idx_v], tok_v, gsem)
        pltpu.sync_copy(pos_hbm.at[pl.ds(prow0, _C)], pos_v)
        gather.wait()

        def add_row(r, carry):
            for k in range(_D // _L):
                sl = pl.ds(k * _L, _L)
                tok_v[r, sl] = tok_v[r, sl] + pos_v[r, sl]
            return carry

        lax.fori_loop(0, _C, add_row, 0)
        pltpu.sync_copy(tok_v, out_hbm.at[pl.ds(row0, _C)])


def kernel(x, token_table, pos_table):
    xf = x.reshape(_BT).astype(jnp.int32)
    out = _emb_lookup(xf, token_table, pos_table)
    return out.reshape(_B, _S, _D)


# speedup vs baseline: 1.5090x; 1.5090x over previous
import functools

import jax
import jax.numpy as jnp
from jax import lax
from jax.experimental import pallas as pl
from jax.experimental.pallas import tpu as pltpu
from jax.experimental.pallas import tpu_sc as plsc

_B = 4
_S = 8192
_D = 128
_C = 128

_info = plsc.get_sparse_core_info()
_NC, _NS, _L = _info.num_cores, _info.num_subcores, _info.num_lanes
_NW = _NC * _NS
_SPW = _S // _NW
_NSS = _SPW // _C
_NBLK = _S // _C

_CHUNKS = [(ss, b) for ss in range(_NSS) for b in range(_B)]


@functools.partial(
    pl.kernel,
    mesh=plsc.VectorSubcoreMesh(core_axis_name="c", subcore_axis_name="s"),
    out_type=jax.ShapeDtypeStruct((_B, _S, _D), jnp.float32),
    scratch_types=[
        pltpu.VMEM((_B, _NSS, _C), jnp.int32),
        pltpu.VMEM((_SPW, _D), jnp.float32),
        pltpu.VMEM((_C, _D), jnp.float32),
        pltpu.VMEM((_C, _D), jnp.float32),
        pltpu.SemaphoreType.DMA,
        pltpu.SemaphoreType.DMA,
        pltpu.SemaphoreType.DMA,
        pltpu.SemaphoreType.DMA,
        pltpu.SemaphoreType.DMA,
        pltpu.SemaphoreType.DMA,
    ],
)
def _emb_lookup(x_hbm, table_hbm, pos_hbm, out_hbm,
                idx_v, pos_v, tok0, tok1,
                isem, psem, g0, g1, st0, st1):
    wid = lax.axis_index("s") * _NC + lax.axis_index("c")
    s_base = wid * _SPW
    blk = wid * _NSS

    icopy = pltpu.async_copy(x_hbm.at[:, pl.ds(blk, _NSS), :], idx_v, isem)
    pcopy = pltpu.async_copy(pos_hbm.at[pl.ds(s_base, _SPW)], pos_v, psem)

    toks = [tok0, tok1]
    gsems = [g0, g1]
    ssems = [st0, st1]
    gathers = [None, None]
    stores = [None, None]

    icopy.wait()
    ss0, b0 = _CHUNKS[0]
    gathers[0] = pltpu.async_copy(table_hbm.at[idx_v.at[b0, ss0]], toks[0], gsems[0])
    pcopy.wait()

    for j, (ss, b) in enumerate(_CHUNKS):
        cur = j % 2
        nxt = 1 - cur
        if j + 1 < len(_CHUNKS):
            ss1, b1 = _CHUNKS[j + 1]
            if stores[nxt] is not None:
                stores[nxt].wait()
            gathers[nxt] = pltpu.async_copy(
                table_hbm.at[idx_v.at[b1, ss1]], toks[nxt], gsems[nxt])
        gathers[cur].wait()
        tok = toks[cur]

        def add_row(r, carry, tok=tok, ss=ss):
            for k in range(_D // _L):
                sl = pl.ds(k * _L, _L)
                plsc.addupdate(tok.at[r, sl], pos_v[ss * _C + r, sl])
            return carry

        lax.fori_loop(0, _C, add_row, 0)
        stores[cur] = pltpu.async_copy(
            tok, out_hbm.at[b, pl.ds(s_base + ss * _C, _C)], ssems[cur])

    stores[0].wait()
    stores[1].wait()


def kernel(x, token_table, pos_table):
    x3 = x.reshape(_B, _NBLK, _C).astype(jnp.int32)
    return _emb_lookup(x3, token_table, pos_table)
